# baseline (speedup 1.0000x reference)
import functools

import jax
import jax.numpy as jnp
from jax import lax
from jax.experimental import pallas as pl
from jax.experimental.pallas import tpu as pltpu
from jax.experimental.pallas import tpu_sc as plsc

N = 10000
NP = 10240
D = 128
E = 320000
CHUNK = 128
NCORE = 2
NSUB = 16
NW = NCORE * NSUB
RPS = NP // NSUB
NCHP = 2560
EP = NCHP * CHUNK
CPS = NCHP // NW
GC = 64
NCHG = EP // GC
CPSG = NCHG // NW
IB = 40
NBUF = 4
PADIDX = N


def _deg_body(dst2d, deg_out, idxblk, hist):
    cid = lax.axis_index("c")
    sid = lax.axis_index("s")
    w = cid * NSUB + sid

    def zstep(i, _):
        hist[pl.ds(i * 16, 16)] = jnp.zeros((16,), jnp.float32)
        return 0
    lax.fori_loop(0, NP // 16, zstep, 0)

    pltpu.sync_copy(dst2d.at[pl.ds(w * CPS, CPS)], idxblk)
    ones16 = jnp.full((16,), 1.0, jnp.float32)

    def step(t, _):
        for g in range(CHUNK // 16):
            idxv = idxblk[t, pl.ds(g * 16, 16)]
            plsc.addupdate_scatter(hist, [idxv], ones16)
        return 0
    lax.fori_loop(0, CPS, step, 0)

    pltpu.sync_copy(hist, deg_out.at[w])


_deg_kernel = functools.partial(
    pl.kernel,
    out_type=jax.ShapeDtypeStruct((NW, NP), jnp.float32),
    mesh=plsc.VectorSubcoreMesh(core_axis_name="c", subcore_axis_name="s"),
    scratch_types=[
        pltpu.VMEM((CPS, CHUNK), jnp.int32),
        pltpu.VMEM((NP,), jnp.float32),
    ],
    compiler_params=pltpu.CompilerParams(needs_layout_passes=False),
)(_deg_body)


def _agg_body(hw, src2d, dst2d, p0, p1, idxs, idxd, msg, acc,
              g0, g1, g2, g3, sc0, sc1, sc2, sc3):
    cid = lax.axis_index("c")
    sid = lax.axis_index("s")
    gsem = (g0, g1, g2, g3)
    ssem = (sc0, sc1, sc2, sc3)

    r0 = sid * RPS
    pltpu.sync_copy(hw.at[pl.ds(r0, RPS)], acc.at[pl.ds(r0, RPS)])
    plsc.subcore_barrier()

    base = cid * (NCHG // NCORE) + sid * CPSG

    def gather(t, b):
        pltpu.async_copy(hw.at[idxs.at[t]], msg.at[b], gsem[b])

    def wait_gather(t, b):
        pltpu.make_async_copy(hw.at[idxs.at[t]], msg.at[b], gsem[b]).wait()

    for hb in range(CPSG // IB):
        pltpu.sync_copy(src2d.at[pl.ds(base + hb * IB, IB)], idxs)
        pltpu.sync_copy(dst2d.at[pl.ds(base + hb * IB, IB)], idxd)

        gather(0, 0)
        gather(1, 1)

        def step(tt, _):
            for b in range(NBUF):
                u = tt * NBUF + b
                bn = (b + 2) % NBUF

                @pl.when(u + 2 < IB)
                def _():
                    @pl.when(u >= 2)
                    def _():
                        pltpu.make_async_copy(msg.at[bn],
                                              acc.at[idxd.at[u - 2]],
                                              ssem[bn]).wait()
                    pltpu.async_copy(hw.at[idxs.at[u + 2]], msg.at[bn],
                                     gsem[bn])

                wait_gather(u, b)
                pltpu.async_copy(msg.at[b], acc.at[idxd.at[u]],
                                 ssem[b], add=True)
            return 0
        lax.fori_loop(0, IB // NBUF, step, 0)

        for u in (IB - 4, IB - 3, IB - 2, IB - 1):
            b = u % NBUF
            pltpu.make_async_copy(msg.at[b], acc.at[idxd.at[u]],
                                  ssem[b]).wait()

    plsc.subcore_barrier()

    @pl.when(cid == 0)
    def _():
        pltpu.sync_copy(acc.at[pl.ds(r0, RPS)], p0.at[pl.ds(r0, RPS)])

    @pl.when(cid == 1)
    def _():
        pltpu.sync_copy(acc.at[pl.ds(r0, RPS)], p1.at[pl.ds(r0, RPS)])


_agg_kernel = functools.partial(
    pl.kernel,
    out_type=(jax.ShapeDtypeStruct((NP, D), jnp.float32),
              jax.ShapeDtypeStruct((NP, D), jnp.float32)),
    mesh=plsc.VectorSubcoreMesh(core_axis_name="c", subcore_axis_name="s"),
    scratch_types=[
        pltpu.VMEM((IB, GC), jnp.int32),
        pltpu.VMEM((IB, GC), jnp.int32),
        pltpu.VMEM((NBUF, GC, D), jnp.float32),
        pltpu.VMEM_SHARED((NP, D), jnp.float32),
        pltpu.SemaphoreType.DMA,
        pltpu.SemaphoreType.DMA,
        pltpu.SemaphoreType.DMA,
        pltpu.SemaphoreType.DMA,
        pltpu.SemaphoreType.DMA,
        pltpu.SemaphoreType.DMA,
        pltpu.SemaphoreType.DMA,
        pltpu.SemaphoreType.DMA,
    ],
)(_agg_body)


def _dinv_from_parts(deg_ref):
    ones = jnp.ones((NW, 1), jnp.float32)
    deg = lax.dot_general(deg_ref[...], ones, (((0,), (0,)), ((), ())),
                          preferred_element_type=jnp.float32)
    return lax.rsqrt(deg[:N, :] + 1.0)


def _tc_l1(x_ref, w_ref, deg_ref, hw_ref):
    dinv = _dinv_from_parts(deg_ref)
    hw = jnp.dot(x_ref[...], w_ref[...], preferred_element_type=jnp.float32)
    hw_ref[:N, :] = hw * dinv
    hw_ref[N:, :] = jnp.zeros((NP - N, D), jnp.float32)


def _tc_mid(p0_ref, p1_ref, hwp_ref, deg_ref, b_ref, g_ref, be_ref, w_ref,
            hw_ref):
    dinv = _dinv_from_parts(deg_ref)
    agg = p0_ref[:N, :] + p1_ref[:N, :] - hwp_ref[:N, :]
    h = agg * dinv + b_ref[...]
    m = jnp.mean(h, axis=0, keepdims=True)
    c = h - m
    v = jnp.mean(c * c, axis=0, keepdims=True)
    h = c * lax.rsqrt(v + 1e-5) * g_ref[...] + be_ref[...]
    h = jnp.maximum(h, 0.0)
    hw = jnp.dot(h, w_ref[...], preferred_element_type=jnp.float32)
    hw_ref[:N, :] = hw * dinv
    hw_ref[N:, :] = jnp.zeros((NP - N, D), jnp.float32)


def _tc_fin(p0_ref, p1_ref, hwp_ref, deg_ref, b_ref, out_ref):
    dinv = _dinv_from_parts(deg_ref)
    agg = p0_ref[:N, :] + p1_ref[:N, :] - hwp_ref[:N, :]
    out_ref[...] = agg * dinv + b_ref[...]


_l1_call = pl.pallas_call(
    _tc_l1,
    out_shape=jax.ShapeDtypeStruct((NP, D), jnp.float32),
)

_mid_call = pl.pallas_call(
    _tc_mid,
    out_shape=jax.ShapeDtypeStruct((NP, D), jnp.float32),
)

_fin_call = pl.pallas_call(
    _tc_fin,
    out_shape=jax.ShapeDtypeStruct((N, D), jnp.float32),
)


@jax.jit
def _run(x, adj_t, W1, b1, W2, b2, W3, b3, g1, be1, g2, be2):
    pad = N + (jnp.arange(EP - E, dtype=jnp.int32) % (NP - N))
    srcp = jnp.concatenate([adj_t[0], pad])
    dstp = jnp.concatenate([adj_t[1], pad])
    src2d = srcp.reshape(NCHG, GC)
    dst2d = dstp.reshape(NCHG, GC)
    dst2d_deg = dstp.reshape(NCHP, CHUNK)
    b1r = b1.reshape(1, D)
    b2r = b2.reshape(1, D)
    b3r = b3.reshape(1, D)
    g1r = g1.reshape(1, D)
    g2r = g2.reshape(1, D)
    be1r = be1.reshape(1, D)
    be2r = be2.reshape(1, D)

    degp = _deg_kernel(dst2d_deg)

    hw = _l1_call(x, W1, degp)
    p0, p1 = _agg_kernel(hw, src2d, dst2d)
    hw2 = _mid_call(p0, p1, hw, degp, b1r, g1r, be1r, W2)
    p0, p1 = _agg_kernel(hw2, src2d, dst2d)
    hw3 = _mid_call(p0, p1, hw2, degp, b2r, g2r, be2r, W3)
    p0, p1 = _agg_kernel(hw3, src2d, dst2d)
    return _fin_call(p0, p1, hw3, degp, b3r)


def kernel(x, adj_t, W1, b1, W2, b2, W3, b3, g1, be1, g2, be2):
    return _run(x, adj_t, W1, b1, W2, b2, W3, b3, g1, be1, g2, be2)


# speedup vs baseline: 1.0247x; 1.0247x over previous
import functools

import jax
import jax.numpy as jnp
from jax import lax
from jax.experimental import pallas as pl
from jax.experimental.pallas import tpu as pltpu
from jax.experimental.pallas import tpu_sc as plsc

N = 10000
NP = 10240
D = 128
E = 320000
CHUNK = 128
NCORE = 2
NSUB = 16
NW = NCORE * NSUB
RPS = NP // NSUB
NCHP = 2560
EP = NCHP * CHUNK
CPS = NCHP // NW
GC = 128
NCHG = EP // GC
CPSG = NCHG // NW
IB = 40
NBUF = 2
PADIDX = N


def _deg_body(dst2d, deg_out, idxblk, hist):
    cid = lax.axis_index("c")
    sid = lax.axis_index("s")
    w = cid * NSUB + sid

    def zstep(i, _):
        hist[pl.ds(i * 16, 16)] = jnp.zeros((16,), jnp.float32)
        return 0
    lax.fori_loop(0, NP // 16, zstep, 0)

    pltpu.sync_copy(dst2d.at[pl.ds(w * CPS, CPS)], idxblk)
    ones16 = jnp.full((16,), 1.0, jnp.float32)

    def step(t, _):
        for g in range(CHUNK // 16):
            idxv = idxblk[t, pl.ds(g * 16, 16)]
            plsc.addupdate_scatter(hist, [idxv], ones16)
        return 0
    lax.fori_loop(0, CPS, step, 0)

    pltpu.sync_copy(hist, deg_out.at[w])


_deg_kernel = functools.partial(
    pl.kernel,
    out_type=jax.ShapeDtypeStruct((NW, NP), jnp.float32),
    mesh=plsc.VectorSubcoreMesh(core_axis_name="c", subcore_axis_name="s"),
    scratch_types=[
        pltpu.VMEM((CPS, CHUNK), jnp.int32),
        pltpu.VMEM((NP,), jnp.float32),
    ],
    compiler_params=pltpu.CompilerParams(needs_layout_passes=False),
)(_deg_body)


def _agg_body(hw, src2d, dst2d, p0, p1, idxs, idxd, msg, acc,
              g0, g1, sc0, sc1):
    cid = lax.axis_index("c")
    sid = lax.axis_index("s")
    gsem = (g0, g1)
    ssem = (sc0, sc1)

    r0 = sid * RPS
    pltpu.sync_copy(hw.at[pl.ds(r0, RPS)], acc.at[pl.ds(r0, RPS)])
    plsc.subcore_barrier()

    base = cid * (NCHG // NCORE) + sid * CPSG

    def gather(t, b):
        pltpu.async_copy(hw.at[idxs.at[t]], msg.at[b], gsem[b])

    def wait_gather(t, b):
        pltpu.make_async_copy(hw.at[idxs.at[t]], msg.at[b], gsem[b]).wait()

    for hb in range(CPSG // IB):
        pltpu.sync_copy(src2d.at[pl.ds(base + hb * IB, IB)], idxs)
        pltpu.sync_copy(dst2d.at[pl.ds(base + hb * IB, IB)], idxd)

        gather(0, 0)

        def step(tt, _):
            for b in range(NBUF):
                u = tt * NBUF + b
                b2 = (b + 1) % NBUF

                @pl.when(u + 1 < IB)
                def _():
                    @pl.when(u >= 1)
                    def _():
                        pltpu.make_async_copy(msg.at[b2],
                                              acc.at[idxd.at[u - 1]],
                                              ssem[b2]).wait()
                    pltpu.async_copy(hw.at[idxs.at[u + 1]], msg.at[b2],
                                     gsem[b2])

                wait_gather(u, b)
                pltpu.async_copy(msg.at[b], acc.at[idxd.at[u]],
                                 ssem[b], add=True)
            return 0
        lax.fori_loop(0, IB // NBUF, step, 0)

        for u in (IB - 2, IB - 1):
            b = u % NBUF
            pltpu.make_async_copy(msg.at[b], acc.at[idxd.at[u]],
                                  ssem[b]).wait()

    plsc.subcore_barrier()

    @pl.when(cid == 0)
    def _():
        pltpu.sync_copy(acc.at[pl.ds(r0, RPS)], p0.at[pl.ds(r0, RPS)])

    @pl.when(cid == 1)
    def _():
        pltpu.sync_copy(acc.at[pl.ds(r0, RPS)], p1.at[pl.ds(r0, RPS)])


_agg_kernel = functools.partial(
    pl.kernel,
    out_type=(jax.ShapeDtypeStruct((NP, D), jnp.float32),
              jax.ShapeDtypeStruct((NP, D), jnp.float32)),
    mesh=plsc.VectorSubcoreMesh(core_axis_name="c", subcore_axis_name="s"),
    scratch_types=[
        pltpu.VMEM((IB, GC), jnp.int32),
        pltpu.VMEM((IB, GC), jnp.int32),
        pltpu.VMEM((NBUF, GC, D), jnp.float32),
        pltpu.VMEM_SHARED((NP, D), jnp.float32),
        pltpu.SemaphoreType.DMA,
        pltpu.SemaphoreType.DMA,
        pltpu.SemaphoreType.DMA,
        pltpu.SemaphoreType.DMA,
    ],
)(_agg_body)


def _dinv_from_parts(deg_ref):
    ones = jnp.ones((NW, 1), jnp.float32)
    deg = lax.dot_general(deg_ref[...], ones, (((0,), (0,)), ((), ())),
                          preferred_element_type=jnp.float32)
    return lax.rsqrt(deg[:N, :] + 1.0)


def _tc_l1(x_ref, w_ref, deg_ref, hw_ref):
    dinv = _dinv_from_parts(deg_ref)
    hw = jnp.dot(x_ref[...], w_ref[...], preferred_element_type=jnp.float32)
    hw_ref[:N, :] = hw * dinv
    hw_ref[N:, :] = jnp.zeros((NP - N, D), jnp.float32)


def _tc_mid(p0_ref, p1_ref, hwp_ref, deg_ref, b_ref, g_ref, be_ref, w_ref,
            hw_ref):
    dinv = _dinv_from_parts(deg_ref)
    agg = p0_ref[:N, :] + p1_ref[:N, :] - hwp_ref[:N, :]
    h = agg * dinv + b_ref[...]
    m = jnp.mean(h, axis=0, keepdims=True)
    c = h - m
    v = jnp.mean(c * c, axis=0, keepdims=True)
    h = c * lax.rsqrt(v + 1e-5) * g_ref[...] + be_ref[...]
    h = jnp.maximum(h, 0.0)
    hw = jnp.dot(h, w_ref[...], preferred_element_type=jnp.float32)
    hw_ref[:N, :] = hw * dinv
    hw_ref[N:, :] = jnp.zeros((NP - N, D), jnp.float32)


def _tc_fin(p0_ref, p1_ref, hwp_ref, deg_ref, b_ref, out_ref):
    dinv = _dinv_from_parts(deg_ref)
    agg = p0_ref[:N, :] + p1_ref[:N, :] - hwp_ref[:N, :]
    out_ref[...] = agg * dinv + b_ref[...]


_l1_call = pl.pallas_call(
    _tc_l1,
    out_shape=jax.ShapeDtypeStruct((NP, D), jnp.float32),
)

_mid_call = pl.pallas_call(
    _tc_mid,
    out_shape=jax.ShapeDtypeStruct((NP, D), jnp.float32),
)

_fin_call = pl.pallas_call(
    _tc_fin,
    out_shape=jax.ShapeDtypeStruct((N, D), jnp.float32),
)


@jax.jit
def _run(x, adj_t, W1, b1, W2, b2, W3, b3, g1, be1, g2, be2):
    pad = N + (jnp.arange(EP - E, dtype=jnp.int32) % (NP - N))
    srcp = jnp.concatenate([adj_t[0], pad])
    dstp = jnp.concatenate([adj_t[1], pad])
    src2d = srcp.reshape(NCHG, GC)
    dst2d = dstp.reshape(NCHG, GC)
    dst2d_deg = dstp.reshape(NCHP, CHUNK)
    b1r = b1.reshape(1, D)
    b2r = b2.reshape(1, D)
    b3r = b3.reshape(1, D)
    g1r = g1.reshape(1, D)
    g2r = g2.reshape(1, D)
    be1r = be1.reshape(1, D)
    be2r = be2.reshape(1, D)

    degp = _deg_kernel(dst2d_deg)

    hw = _l1_call(x, W1, degp)
    p0, p1 = _agg_kernel(hw, src2d, dst2d)
    hw2 = _mid_call(p0, p1, hw, degp, b1r, g1r, be1r, W2)
    p0, p1 = _agg_kernel(hw2, src2d, dst2d)
    hw3 = _mid_call(p0, p1, hw2, degp, b2r, g2r, be2r, W3)
    p0, p1 = _agg_kernel(hw3, src2d, dst2d)
    return _fin_call(p0, p1, hw3, degp, b3r)


def kernel(x, adj_t, W1, b1, W2, b2, W3, b3, g1, be1, g2, be2):
    return _run(x, adj_t, W1, b1, W2, b2, W3, b3, g1, be1, g2, be2)


# speedup vs baseline: 1.0344x; 1.0094x over previous
import functools

import jax
import jax.numpy as jnp
from jax import lax
from jax.experimental import pallas as pl
from jax.experimental.pallas import tpu as pltpu
from jax.experimental.pallas import tpu_sc as plsc

N = 10000
NP = 10240
D = 128
E = 320000
CHUNK = 128
NCORE = 2
NSUB = 16
NW = NCORE * NSUB
RPS = NP // NSUB
NCHP = 2560
EP = NCHP * CHUNK
CPS = NCHP // NW
GC = 128
NCHG = EP // GC
CPSG = NCHG // NW
IB = 40
NBUF = 2
PADIDX = N


def _deg_body(dst2d, deg_out, idxblk, hist):
    cid = lax.axis_index("c")
    sid = lax.axis_index("s")
    w = cid * NSUB + sid

    def zstep(i, _):
        hist[pl.ds(i * 16, 16)] = jnp.zeros((16,), jnp.float32)
        return 0
    lax.fori_loop(0, NP // 16, zstep, 0)

    pltpu.sync_copy(dst2d.at[pl.ds(w * CPS, CPS)], idxblk)
    ones16 = jnp.full((16,), 1.0, jnp.float32)

    def step(t, _):
        for g in range(CHUNK // 16):
            idxv = idxblk[t, pl.ds(g * 16, 16)]
            plsc.addupdate_scatter(hist, [idxv], ones16)
        return 0
    lax.fori_loop(0, CPS, step, 0)

    pltpu.sync_copy(hist, deg_out.at[w])


_deg_kernel = functools.partial(
    pl.kernel,
    out_type=jax.ShapeDtypeStruct((NW, NP), jnp.float32),
    mesh=plsc.VectorSubcoreMesh(core_axis_name="c", subcore_axis_name="s"),
    scratch_types=[
        pltpu.VMEM((CPS, CHUNK), jnp.int32),
        pltpu.VMEM((NP,), jnp.float32),
    ],
    compiler_params=pltpu.CompilerParams(needs_layout_passes=False),
)(_deg_body)


def _agg_body(hw, src2d, dst2d, p0, p1, idxs, idxd, msg, acc,
              g0, g1, sc0, sc1, zsem):
    cid = lax.axis_index("c")
    sid = lax.axis_index("s")
    gsem = (g0, g1)
    ssem = (sc0, sc1)

    r0 = sid * RPS

    @pl.when(cid == 0)
    def _():
        pltpu.sync_copy(hw.at[pl.ds(r0, RPS)], acc.at[pl.ds(r0, RPS)])

    @pl.when(cid == 1)
    def _():
        for off, sz in ((0, 240), (240, 240), (480, 160)):
            pltpu.async_copy(hw.at[pl.ds(N, sz)],
                             acc.at[pl.ds(r0 + off, sz)], zsem)
        for off, sz in ((0, 240), (240, 240), (480, 160)):
            pltpu.make_async_copy(hw.at[pl.ds(N, sz)],
                                  acc.at[pl.ds(r0 + off, sz)], zsem).wait()
    plsc.subcore_barrier()

    base = cid * (NCHG // NCORE) + sid * CPSG

    def gather(t, b):
        pltpu.async_copy(hw.at[idxs.at[t]], msg.at[b], gsem[b])

    def wait_gather(t, b):
        pltpu.make_async_copy(hw.at[idxs.at[t]], msg.at[b], gsem[b]).wait()

    for hb in range(CPSG // IB):
        pltpu.sync_copy(src2d.at[pl.ds(base + hb * IB, IB)], idxs)
        pltpu.sync_copy(dst2d.at[pl.ds(base + hb * IB, IB)], idxd)

        gather(0, 0)

        def step(tt, _):
            for b in range(NBUF):
                u = tt * NBUF + b
                b2 = (b + 1) % NBUF

                @pl.when(u + 1 < IB)
                def _():
                    @pl.when(u >= 1)
                    def _():
                        pltpu.make_async_copy(msg.at[b2],
                                              acc.at[idxd.at[u - 1]],
                                              ssem[b2]).wait()
                    pltpu.async_copy(hw.at[idxs.at[u + 1]], msg.at[b2],
                                     gsem[b2])

                wait_gather(u, b)
                pltpu.async_copy(msg.at[b], acc.at[idxd.at[u]],
                                 ssem[b], add=True)
            return 0
        lax.fori_loop(0, IB // NBUF, step, 0)

        for u in (IB - 2, IB - 1):
            b = u % NBUF
            pltpu.make_async_copy(msg.at[b], acc.at[idxd.at[u]],
                                  ssem[b]).wait()

    plsc.subcore_barrier()

    @pl.when(cid == 0)
    def _():
        pltpu.sync_copy(acc.at[pl.ds(r0, RPS)], p0.at[pl.ds(r0, RPS)])

    @pl.when(cid == 1)
    def _():
        pltpu.sync_copy(acc.at[pl.ds(r0, RPS)], p1.at[pl.ds(r0, RPS)])


_agg_kernel = functools.partial(
    pl.kernel,
    out_type=(jax.ShapeDtypeStruct((NP, D), jnp.float32),
              jax.ShapeDtypeStruct((NP, D), jnp.float32)),
    mesh=plsc.VectorSubcoreMesh(core_axis_name="c", subcore_axis_name="s"),
    scratch_types=[
        pltpu.VMEM((IB, GC), jnp.int32),
        pltpu.VMEM((IB, GC), jnp.int32),
        pltpu.VMEM((NBUF, GC, D), jnp.float32),
        pltpu.VMEM_SHARED((NP, D), jnp.float32),
        pltpu.SemaphoreType.DMA,
        pltpu.SemaphoreType.DMA,
        pltpu.SemaphoreType.DMA,
        pltpu.SemaphoreType.DMA,
        pltpu.SemaphoreType.DMA,
    ],
)(_agg_body)


def _dinv_from_parts(deg_ref):
    ones = jnp.ones((NW, 1), jnp.float32)
    deg = lax.dot_general(deg_ref[...], ones, (((0,), (0,)), ((), ())),
                          preferred_element_type=jnp.float32)
    return lax.rsqrt(deg[:N, :] + 1.0)


def _tc_l1(x_ref, w_ref, deg_ref, hw_ref):
    dinv = _dinv_from_parts(deg_ref)
    hw = jnp.dot(x_ref[...], w_ref[...], preferred_element_type=jnp.float32)
    hw_ref[:N, :] = hw * dinv
    hw_ref[N:, :] = jnp.zeros((NP - N, D), jnp.float32)


def _tc_mid(p0_ref, p1_ref, deg_ref, b_ref, g_ref, be_ref, w_ref,
            hw_ref):
    dinv = _dinv_from_parts(deg_ref)
    agg = p0_ref[:N, :] + p1_ref[:N, :]
    h = agg * dinv + b_ref[...]
    m = jnp.mean(h, axis=0, keepdims=True)
    c = h - m
    v = jnp.mean(c * c, axis=0, keepdims=True)
    h = c * lax.rsqrt(v + 1e-5) * g_ref[...] + be_ref[...]
    h = jnp.maximum(h, 0.0)
    hw = jnp.dot(h, w_ref[...], preferred_element_type=jnp.float32)
    hw_ref[:N, :] = hw * dinv
    hw_ref[N:, :] = jnp.zeros((NP - N, D), jnp.float32)


def _tc_fin(p0_ref, p1_ref, deg_ref, b_ref, out_ref):
    dinv = _dinv_from_parts(deg_ref)
    agg = p0_ref[:N, :] + p1_ref[:N, :]
    out_ref[...] = agg * dinv + b_ref[...]


_l1_call = pl.pallas_call(
    _tc_l1,
    out_shape=jax.ShapeDtypeStruct((NP, D), jnp.float32),
)

_mid_call = pl.pallas_call(
    _tc_mid,
    out_shape=jax.ShapeDtypeStruct((NP, D), jnp.float32),
)

_fin_call = pl.pallas_call(
    _tc_fin,
    out_shape=jax.ShapeDtypeStruct((N, D), jnp.float32),
)


@jax.jit
def _run(x, adj_t, W1, b1, W2, b2, W3, b3, g1, be1, g2, be2):
    pad = N + (jnp.arange(EP - E, dtype=jnp.int32) % (NP - N))
    srcp = jnp.concatenate([adj_t[0], pad])
    dstp = jnp.concatenate([adj_t[1], pad])
    src2d = srcp.reshape(NCHG, GC)
    dst2d = dstp.reshape(NCHG, GC)
    dst2d_deg = dstp.reshape(NCHP, CHUNK)
    b1r = b1.reshape(1, D)
    b2r = b2.reshape(1, D)
    b3r = b3.reshape(1, D)
    g1r = g1.reshape(1, D)
    g2r = g2.reshape(1, D)
    be1r = be1.reshape(1, D)
    be2r = be2.reshape(1, D)

    degp = _deg_kernel(dst2d_deg)

    hw = _l1_call(x, W1, degp)
    p0, p1 = _agg_kernel(hw, src2d, dst2d)
    hw2 = _mid_call(p0, p1, degp, b1r, g1r, be1r, W2)
    p0, p1 = _agg_kernel(hw2, src2d, dst2d)
    hw3 = _mid_call(p0, p1, degp, b2r, g2r, be2r, W3)
    p0, p1 = _agg_kernel(hw3, src2d, dst2d)
    return _fin_call(p0, p1, degp, b3r)


def kernel(x, adj_t, W1, b1, W2, b2, W3, b3, g1, be1, g2, be2):
    return _run(x, adj_t, W1, b1, W2, b2, W3, b3, g1, be1, g2, be2)
